# Initial kernel scaffold; baseline (speedup 1.0000x reference)
import jax
import jax.numpy as jnp
from jax.experimental import pallas as pl


def kernel(x_target, x_context, edge_index_tt, edge_index_ct, Wt1, bt1, Wt2, bt2, Wt3, bt3, Wc1, bc1, Wc2, bc2, Wl_tt1, bl_tt1, Wr_tt1, Wl_ct1, bl_ct1, Wr_ct1, gamma, beta, Wl_tt2, bl_tt2, Wr_tt2):
    raise NotImplementedError("write your pallas kernel here")


# speedup vs baseline: 1.2423x; 1.2423x over previous
import functools

import jax
import jax.numpy as jnp
from jax import lax
from jax.experimental import pallas as pl
from jax.experimental.pallas import tpu as pltpu
from jax.experimental.pallas import tpu_sc as plsc

N_NODES = 50000
D = 128
E = 300000

NC = 2
NS = 16

KB = 128
NB = 2352
E_PAD = NB * KB
BATCHES_PER_TILE = NB // NS

NCHUNK = 4
CH = N_NODES // NCHUNK
PT = 784
CHP = NS * PT
NP_OUT = NCHUNK * CHP

BR = 1000


def _mlp_body(nw, x_ref, *refs):
    h = x_ref[...]
    for i in range(nw):
        w = refs[2 * i][...]
        b = refs[2 * i + 1][...]
        h = jnp.maximum(jnp.dot(h, w, preferred_element_type=jnp.float32) + b, 0.0)
    refs[-1][...] = h


def _mlp(x, *wb):
    nw = len(wb) // 2
    n = x.shape[0]
    grid = n // BR
    in_specs = [pl.BlockSpec((BR, D), lambda i: (i, 0))]
    args = [x]
    for i in range(nw):
        in_specs.append(pl.BlockSpec((D, D), lambda i: (0, 0)))
        in_specs.append(pl.BlockSpec((1, D), lambda i: (0, 0)))
        args.append(wb[2 * i])
        args.append(wb[2 * i + 1].reshape(1, D))
    return pl.pallas_call(
        functools.partial(_mlp_body, nw),
        grid=(grid,),
        in_specs=in_specs,
        out_specs=pl.BlockSpec((BR, D), lambda i: (i, 0)),
        out_shape=jax.ShapeDtypeStruct((n, D), jnp.float32),
        compiler_params=pltpu.CompilerParams(dimension_semantics=("parallel",)),
    )(*args)


def _combine_body(s_tt_ref, c_tt_ref, s_ct_ref, c_ct_ref, x_ref,
                  wl1_ref, bl1_ref, wr1_ref, wl2_ref, bl2_ref, wr2_ref,
                  g_ref, b_ref, o_ref):
    m_tt = s_tt_ref[...] / jnp.maximum(c_tt_ref[...][:, 0:1], 1.0)
    m_ct = s_ct_ref[...] / jnp.maximum(c_ct_ref[...][:, 0:1], 1.0)
    x = x_ref[...]
    h = (jnp.dot(m_tt, wl1_ref[...], preferred_element_type=jnp.float32) + bl1_ref[...]
         + jnp.dot(x, wr1_ref[...], preferred_element_type=jnp.float32)
         + jnp.dot(m_ct, wl2_ref[...], preferred_element_type=jnp.float32) + bl2_ref[...]
         + jnp.dot(x, wr2_ref[...], preferred_element_type=jnp.float32))
    h = jnp.maximum(h, 0.0)
    mu = jnp.mean(h, axis=-1, keepdims=True)
    var = jnp.mean((h - mu) ** 2, axis=-1, keepdims=True)
    hn = (h - mu) * lax.rsqrt(var + 1e-5) * g_ref[...] + b_ref[...]
    o_ref[...] = hn + x


def _combine(s_tt, c_tt, s_ct, c_ct, x, wl1, bl1, wr1, wl2, bl2, wr2, g, b):
    full = lambda r, c: pl.BlockSpec((r, c), lambda i: (0, 0))
    blk = lambda r, c: pl.BlockSpec((r, c), lambda i: (i, 0))
    return pl.pallas_call(
        _combine_body,
        grid=(N_NODES // BR,),
        in_specs=[blk(BR, D), blk(BR, D), blk(BR, D), blk(BR, D), blk(BR, D),
                  full(D, D), full(1, D), full(D, D), full(D, D), full(1, D),
                  full(D, D), full(1, D), full(1, D)],
        out_specs=blk(BR, D),
        out_shape=jax.ShapeDtypeStruct((N_NODES, D), jnp.float32),
        compiler_params=pltpu.CompilerParams(dimension_semantics=("parallel",)),
    )(s_tt, c_tt, s_ct, c_ct, x, wl1, bl1.reshape(1, D), wr1, wl2,
      bl2.reshape(1, D), wr2, g.reshape(1, D), b.reshape(1, D))


def _final_body(s_ref, c_ref, x_ref, wl_ref, bl_ref, wr_ref, o_ref):
    m = s_ref[...] / jnp.maximum(c_ref[...][:, 0:1], 1.0)
    x = x_ref[...]
    h = (jnp.dot(m, wl_ref[...], preferred_element_type=jnp.float32) + bl_ref[...]
         + jnp.dot(x, wr_ref[...], preferred_element_type=jnp.float32))
    o_ref[...] = jnp.maximum(h, 0.0) + x


def _final(s, c, x, wl, bl, wr):
    full = lambda r, cc: pl.BlockSpec((r, cc), lambda i: (0, 0))
    blk = lambda r, cc: pl.BlockSpec((r, cc), lambda i: (i, 0))
    return pl.pallas_call(
        _final_body,
        grid=(N_NODES // BR,),
        in_specs=[blk(BR, D), blk(BR, D), blk(BR, D),
                  full(D, D), full(1, D), full(D, D)],
        out_specs=blk(BR, D),
        out_shape=jax.ShapeDtypeStruct((N_NODES, D), jnp.float32),
        compiler_params=pltpu.CompilerParams(dimension_semantics=("parallel",)),
    )(s, c, x, wl, bl.reshape(1, D), wr)


def _agg_body(with_counts, src_hbm, dst_hbm, x_hbm, z128_hbm, ones_hbm,
              *refs):
    if with_counts:
        sums_hbm, cnt_hbm, acc, sidx, didx, rel, rows, sem = refs
    else:
        sums_hbm, acc, sidx, didx, rel, rows, sem = refs
        cnt_hbm = None
    c = lax.axis_index("c")
    s = lax.axis_index("s")
    for t in range(NCHUNK // NC):
        chunk = c + NC * t
        chunk_base = chunk * CH
        my_base = s * PT
        out_base = chunk * CHP + my_base

        def compute_rel(didx_ref, rel_ref):
            for j in range(KB // 16):
                d16 = didx_ref[pl.ds(j * 16, 16)]
                r16 = d16 - chunk_base
                m16 = (r16 >= 0) & (r16 < CH)
                rel_ref[pl.ds(j * 16, 16)] = jnp.where(m16, r16, CH)

        pltpu.sync_copy(z128_hbm, acc.at[pl.ds(my_base, PT)])
        plsc.subcore_barrier()

        def batch_a(k, _):
            b = s + NS * k
            off = b * KB
            pltpu.sync_copy(src_hbm.at[pl.ds(off, KB)], sidx)
            pltpu.sync_copy(dst_hbm.at[pl.ds(off, KB)], didx)
            gcopy = pltpu.async_copy(x_hbm.at[sidx], rows, sem)
            compute_rel(didx, rel)
            gcopy.wait()
            pltpu.sync_copy(rows, acc.at[rel], add=True)
            return 0

        lax.fori_loop(0, BATCHES_PER_TILE, batch_a, 0)
        plsc.subcore_barrier()
        pltpu.sync_copy(acc.at[pl.ds(my_base, PT)],
                        sums_hbm.at[pl.ds(out_base, PT)])
        plsc.subcore_barrier()

        if with_counts:
            pltpu.sync_copy(z128_hbm, acc.at[pl.ds(my_base, PT)])
            pltpu.sync_copy(ones_hbm, rows)
            plsc.subcore_barrier()

            def batch_b(k, _):
                b = s + NS * k
                off = b * KB
                pltpu.sync_copy(dst_hbm.at[pl.ds(off, KB)], didx)
                compute_rel(didx, rel)
                pltpu.sync_copy(rows, acc.at[rel], add=True)
                return 0

            lax.fori_loop(0, BATCHES_PER_TILE, batch_b, 0)
            plsc.subcore_barrier()
            pltpu.sync_copy(acc.at[pl.ds(my_base, PT)],
                            cnt_hbm.at[pl.ds(out_base, PT)])
            plsc.subcore_barrier()


def _agg_call(with_counts):
    n_out = 2 if with_counts else 1
    return pl.kernel(
        functools.partial(_agg_body, with_counts),
        out_type=[jax.ShapeDtypeStruct((NP_OUT, D), jnp.float32)] * n_out,
        mesh=plsc.VectorSubcoreMesh(core_axis_name="c", subcore_axis_name="s"),
        scratch_types=[
            pltpu.VMEM_SHARED((CHP, D), jnp.float32),
            pltpu.VMEM((KB,), jnp.int32),
            pltpu.VMEM((KB,), jnp.int32),
            pltpu.VMEM((KB,), jnp.int32),
            pltpu.VMEM((KB, D), jnp.float32),
            pltpu.SemaphoreType.DMA,
        ],
    )


def _unpad(a):
    return a.reshape(NCHUNK, CHP, D)[:, :CH].reshape(N_NODES, D)


def _agg(ei, x, with_counts=True):
    src = jnp.concatenate([ei[0], jnp.zeros((E_PAD - E,), jnp.int32)])
    dst = jnp.concatenate([ei[1], jnp.full((E_PAD - E,), N_NODES, jnp.int32)])
    z128 = jnp.zeros((PT, D), jnp.float32)
    ones = jnp.ones((KB, D), jnp.float32)
    outs = _agg_call(with_counts)(src, dst, x, z128, ones)
    if with_counts:
        return _unpad(outs[0]), _unpad(outs[1])
    return _unpad(outs[0])


def kernel(x_target, x_context, edge_index_tt, edge_index_ct,
           Wt1, bt1, Wt2, bt2, Wt3, bt3,
           Wc1, bc1, Wc2, bc2,
           Wl_tt1, bl_tt1, Wr_tt1,
           Wl_ct1, bl_ct1, Wr_ct1,
           gamma, beta,
           Wl_tt2, bl_tt2, Wr_tt2):
    xt = _mlp(x_target, Wt1, bt1, Wt2, bt2, Wt3, bt3)
    xc = _mlp(x_context, Wc1, bc1, Wc2, bc2)
    s_tt, c_tt = _agg(edge_index_tt, xt)
    s_ct, c_ct = _agg(edge_index_ct, xc)
    xt1 = _combine(s_tt, c_tt, s_ct, c_ct, xt,
                   Wl_tt1, bl_tt1, Wr_tt1, Wl_ct1, bl_ct1, Wr_ct1, gamma, beta)
    s2, c2 = _agg(edge_index_tt, xt1)
    return _final(s2, c2, xt1, Wl_tt2, bl_tt2, Wr_tt2)


# speedup vs baseline: 1.4150x; 1.1391x over previous
import functools

import jax
import jax.numpy as jnp
from jax import lax
from jax.experimental import pallas as pl
from jax.experimental.pallas import tpu as pltpu
from jax.experimental.pallas import tpu_sc as plsc

N_NODES = 50000
D = 128
E = 300000

NC = 2
NS = 16

KB = 128
NB = 2352
E_PAD = NB * KB
BATCHES_PER_TILE = NB // NS

NCHUNK = 4
CH = N_NODES // NCHUNK
PT = 784
CHP = NS * PT
NP_OUT = NCHUNK * CHP

BR = 1000


def _mlp_body(nw, x_ref, *refs):
    h = x_ref[...]
    for i in range(nw):
        w = refs[2 * i][...]
        b = refs[2 * i + 1][...]
        h = jnp.maximum(jnp.dot(h, w, preferred_element_type=jnp.float32) + b, 0.0)
    refs[-1][...] = h


def _mlp(x, *wb):
    nw = len(wb) // 2
    n = x.shape[0]
    grid = n // BR
    in_specs = [pl.BlockSpec((BR, D), lambda i: (i, 0))]
    args = [x]
    for i in range(nw):
        in_specs.append(pl.BlockSpec((D, D), lambda i: (0, 0)))
        in_specs.append(pl.BlockSpec((1, D), lambda i: (0, 0)))
        args.append(wb[2 * i])
        args.append(wb[2 * i + 1].reshape(1, D))
    return pl.pallas_call(
        functools.partial(_mlp_body, nw),
        grid=(grid,),
        in_specs=in_specs,
        out_specs=pl.BlockSpec((BR, D), lambda i: (i, 0)),
        out_shape=jax.ShapeDtypeStruct((n, D), jnp.float32),
        compiler_params=pltpu.CompilerParams(dimension_semantics=("parallel",)),
    )(*args)


def _combine_body(s_tt_ref, c_tt_ref, s_ct_ref, c_ct_ref, x_ref,
                  wl1_ref, bl1_ref, wr1_ref, wl2_ref, bl2_ref, wr2_ref,
                  g_ref, b_ref, o_ref):
    m_tt = s_tt_ref[...] / jnp.maximum(c_tt_ref[...][:, 0:1], 1.0)
    m_ct = s_ct_ref[...] / jnp.maximum(c_ct_ref[...][:, 0:1], 1.0)
    x = x_ref[...]
    h = (jnp.dot(m_tt, wl1_ref[...], preferred_element_type=jnp.float32) + bl1_ref[...]
         + jnp.dot(x, wr1_ref[...], preferred_element_type=jnp.float32)
         + jnp.dot(m_ct, wl2_ref[...], preferred_element_type=jnp.float32) + bl2_ref[...]
         + jnp.dot(x, wr2_ref[...], preferred_element_type=jnp.float32))
    h = jnp.maximum(h, 0.0)
    mu = jnp.mean(h, axis=-1, keepdims=True)
    var = jnp.mean((h - mu) ** 2, axis=-1, keepdims=True)
    hn = (h - mu) * lax.rsqrt(var + 1e-5) * g_ref[...] + b_ref[...]
    o_ref[...] = hn + x


def _combine(s_tt, c_tt, s_ct, c_ct, x, wl1, bl1, wr1, wl2, bl2, wr2, g, b):
    full = lambda r, c: pl.BlockSpec((r, c), lambda i: (0, 0))
    blk = lambda r, c: pl.BlockSpec((r, c), lambda i: (i, 0))
    return pl.pallas_call(
        _combine_body,
        grid=(N_NODES // BR,),
        in_specs=[blk(BR, D), blk(BR, 16), blk(BR, D), blk(BR, 16), blk(BR, D),
                  full(D, D), full(1, D), full(D, D), full(D, D), full(1, D),
                  full(D, D), full(1, D), full(1, D)],
        out_specs=blk(BR, D),
        out_shape=jax.ShapeDtypeStruct((N_NODES, D), jnp.float32),
        compiler_params=pltpu.CompilerParams(dimension_semantics=("parallel",)),
    )(s_tt, c_tt, s_ct, c_ct, x, wl1, bl1.reshape(1, D), wr1, wl2,
      bl2.reshape(1, D), wr2, g.reshape(1, D), b.reshape(1, D))


def _final_body(s_ref, c_ref, x_ref, wl_ref, bl_ref, wr_ref, o_ref):
    m = s_ref[...] / jnp.maximum(c_ref[...][:, 0:1], 1.0)
    x = x_ref[...]
    h = (jnp.dot(m, wl_ref[...], preferred_element_type=jnp.float32) + bl_ref[...]
         + jnp.dot(x, wr_ref[...], preferred_element_type=jnp.float32))
    o_ref[...] = jnp.maximum(h, 0.0) + x


def _final(s, c, x, wl, bl, wr):
    full = lambda r, cc: pl.BlockSpec((r, cc), lambda i: (0, 0))
    blk = lambda r, cc: pl.BlockSpec((r, cc), lambda i: (i, 0))
    return pl.pallas_call(
        _final_body,
        grid=(N_NODES // BR,),
        in_specs=[blk(BR, D), blk(BR, 16), blk(BR, D),
                  full(D, D), full(1, D), full(D, D)],
        out_specs=blk(BR, D),
        out_shape=jax.ShapeDtypeStruct((N_NODES, D), jnp.float32),
        compiler_params=pltpu.CompilerParams(dimension_semantics=("parallel",)),
    )(s, c, x, wl, bl.reshape(1, D), wr)


def _compute_rel(didx_ref, rel_ref, chunk_base):
    for j in range(KB // 16):
        d16 = didx_ref[pl.ds(j * 16, 16)]
        r16 = d16 - chunk_base
        m16 = (r16 >= 0) & (r16 < CH)
        rel_ref[pl.ds(j * 16, 16)] = jnp.where(m16, r16, CH)


def _agg_sum_body(src_hbm, dst_hbm, x_hbm, z128_hbm,
                  sums_hbm, acc, sidx, didx, rel, rows, sem):
    c = lax.axis_index("c")
    s = lax.axis_index("s")
    for t in range(NCHUNK // NC):
        chunk = c + NC * t
        chunk_base = chunk * CH
        my_base = s * PT
        out_base = chunk * CHP + my_base

        pltpu.sync_copy(z128_hbm, acc.at[pl.ds(my_base, PT)])
        plsc.subcore_barrier()

        def batch_a(k, _):
            b = s + NS * k
            off = b * KB
            pltpu.sync_copy(src_hbm.at[pl.ds(off, KB)], sidx)
            pltpu.sync_copy(dst_hbm.at[pl.ds(off, KB)], didx)
            gcopy = pltpu.async_copy(x_hbm.at[sidx], rows, sem)
            _compute_rel(didx, rel, chunk_base)
            gcopy.wait()
            pltpu.sync_copy(rows, acc.at[rel], add=True)
            return 0

        lax.fori_loop(0, BATCHES_PER_TILE, batch_a, 0)
        plsc.subcore_barrier()
        pltpu.sync_copy(acc.at[pl.ds(my_base, PT)],
                        sums_hbm.at[pl.ds(out_base, PT)])
        plsc.subcore_barrier()


def _agg_cnt_body(dst_hbm, z16_hbm, ones_hbm,
                  cnt_hbm, accc, didx, rel, onesv):
    c = lax.axis_index("c")
    s = lax.axis_index("s")
    pltpu.sync_copy(ones_hbm, onesv)
    for t in range(NCHUNK // NC):
        chunk = c + NC * t
        chunk_base = chunk * CH
        my_base = s * PT
        out_base = chunk * CHP + my_base

        pltpu.sync_copy(z16_hbm, accc.at[pl.ds(my_base, PT)])
        plsc.subcore_barrier()

        def batch_a(k, _):
            b = s + NS * k
            off = b * KB
            pltpu.sync_copy(dst_hbm.at[pl.ds(off, KB)], didx)
            _compute_rel(didx, rel, chunk_base)
            pltpu.sync_copy(onesv, accc.at[rel], add=True)
            return 0

        lax.fori_loop(0, BATCHES_PER_TILE, batch_a, 0)
        plsc.subcore_barrier()
        pltpu.sync_copy(accc.at[pl.ds(my_base, PT)],
                        cnt_hbm.at[pl.ds(out_base, PT)])
        plsc.subcore_barrier()


def _agg_sum_call():
    return pl.kernel(
        _agg_sum_body,
        out_type=[jax.ShapeDtypeStruct((NP_OUT, D), jnp.float32)],
        mesh=plsc.VectorSubcoreMesh(core_axis_name="c", subcore_axis_name="s"),
        scratch_types=[
            pltpu.VMEM_SHARED((CHP, D), jnp.float32),
            pltpu.VMEM((KB,), jnp.int32),
            pltpu.VMEM((KB,), jnp.int32),
            pltpu.VMEM((KB,), jnp.int32),
            pltpu.VMEM((KB, D), jnp.float32),
            pltpu.SemaphoreType.DMA,
        ],
    )


def _agg_cnt_call():
    return pl.kernel(
        _agg_cnt_body,
        out_type=[jax.ShapeDtypeStruct((NP_OUT, 16), jnp.float32)],
        mesh=plsc.VectorSubcoreMesh(core_axis_name="c", subcore_axis_name="s"),
        scratch_types=[
            pltpu.VMEM_SHARED((CHP, 16), jnp.float32),
            pltpu.VMEM((KB,), jnp.int32),
            pltpu.VMEM((KB,), jnp.int32),
            pltpu.VMEM((KB, 16), jnp.float32),
        ],
    )


def _unpad(a):
    w = a.shape[-1]
    return a.reshape(NCHUNK, CHP, w)[:, :CH].reshape(N_NODES, w)


def _agg_sum(ei, x):
    src = jnp.concatenate([ei[0], jnp.zeros((E_PAD - E,), jnp.int32)])
    dst = jnp.concatenate([ei[1], jnp.full((E_PAD - E,), N_NODES, jnp.int32)])
    z128 = jnp.zeros((PT, D), jnp.float32)
    outs = _agg_sum_call()(src, dst, x, z128)
    return _unpad(outs[0])


def _agg_cnt(ei):
    dst = jnp.concatenate([ei[1], jnp.full((E_PAD - E,), N_NODES, jnp.int32)])
    z16 = jnp.zeros((PT, 16), jnp.float32)
    ones = jnp.ones((KB, 16), jnp.float32)
    outs = _agg_cnt_call()(dst, z16, ones)
    return _unpad(outs[0])


def kernel(x_target, x_context, edge_index_tt, edge_index_ct,
           Wt1, bt1, Wt2, bt2, Wt3, bt3,
           Wc1, bc1, Wc2, bc2,
           Wl_tt1, bl_tt1, Wr_tt1,
           Wl_ct1, bl_ct1, Wr_ct1,
           gamma, beta,
           Wl_tt2, bl_tt2, Wr_tt2):
    xt = _mlp(x_target, Wt1, bt1, Wt2, bt2, Wt3, bt3)
    xc = _mlp(x_context, Wc1, bc1, Wc2, bc2)
    s_tt = _agg_sum(edge_index_tt, xt)
    c_tt = _agg_cnt(edge_index_tt)
    s_ct = _agg_sum(edge_index_ct, xc)
    c_ct = _agg_cnt(edge_index_ct)
    xt1 = _combine(s_tt, c_tt, s_ct, c_ct, xt,
                   Wl_tt1, bl_tt1, Wr_tt1, Wl_ct1, bl_ct1, Wr_ct1, gamma, beta)
    s2 = _agg_sum(edge_index_tt, xt1)
    return _final(s2, c_tt, xt1, Wl_tt2, bl_tt2, Wr_tt2)
